# Initial kernel scaffold; baseline (speedup 1.0000x reference)
import jax
import jax.numpy as jnp
from jax.experimental import pallas as pl


def kernel(x, edge_index, W_in, b_in, taps_W, taps_b, W_out, b_out):
    raise NotImplementedError("write your pallas kernel here")


# speedup vs baseline: 6.7420x; 6.7420x over previous
import functools

import jax
import jax.numpy as jnp
from jax import lax
from jax.experimental import pallas as pl
from jax.experimental.pallas import tpu as pltpu
from jax.experimental.pallas import tpu_sc as plsc

_SLOPE = 0.01


def _leaky(v):
    return jnp.where(v >= 0, v, _SLOPE * v)


@functools.lru_cache(maxsize=None)
def _make_shift(N, E, D):
    try:
        info = plsc.get_sparse_core_info()
        NC, NS = info.num_cores, info.num_subcores
    except ValueError:
        NC, NS = 2, 16
    NW = NC * NS
    assert E % NW == 0
    EPW = E // NW
    CH = 128
    n_full = EPW // CH
    tail = EPW - n_full * CH
    assert N % NS == 0
    RPT = N // NS
    assert (RPT * D) % 8 == 0 and (EPW % 8 == 0) and (tail % 8 == 0)

    mesh = plsc.VectorSubcoreMesh(core_axis_name="c", subcore_axis_name="s",
                                  num_cores=NC, num_subcores=NS)

    TL = max(tail, 8)
    scratch = [
        pltpu.VMEM((CH,), jnp.int32),
        pltpu.VMEM((CH,), jnp.int32),
        pltpu.VMEM((CH, D), jnp.float32),
        pltpu.VMEM((TL,), jnp.int32),
        pltpu.VMEM((TL,), jnp.int32),
        pltpu.VMEM((TL, D), jnp.float32),
        pltpu.VMEM_SHARED((N, D), jnp.float32),
        pltpu.SemaphoreType.DMA,
    ]

    @functools.partial(
        pl.kernel,
        out_type=jax.ShapeDtypeStruct((NC, N, D), jnp.float32),
        mesh=mesh,
        scratch_types=scratch,
        compiler_params=pltpu.CompilerParams(use_tc_tiling_on_sc=False),
    )
    def shift(z_hbm, src_hbm, dst_hbm, zeros_hbm, part_hbm, idx_s, idx_d,
              rows, t_idx_s, t_idx_d, t_rows, acc, sem):
        c = lax.axis_index("c")
        s = lax.axis_index("s")
        wid = c * NS + s
        @pl.when(s == 0)
        def _():
            pltpu.sync_copy(zeros_hbm, acc)
        plsc.subcore_barrier()

        base = wid * EPW

        def body(j, carry):
            off = base + j * CH
            pltpu.sync_copy(src_hbm.at[pl.ds(off, CH)], idx_s)
            pltpu.sync_copy(dst_hbm.at[pl.ds(off, CH)], idx_d)
            pltpu.async_copy(z_hbm.at[idx_s], rows, sem).wait()
            pltpu.sync_copy(rows, acc.at[idx_d], add=True)
            return carry

        lax.fori_loop(0, n_full, body, 0)

        if tail:
            off = base + n_full * CH
            pltpu.sync_copy(src_hbm.at[pl.ds(off, tail)], t_idx_s)
            pltpu.sync_copy(dst_hbm.at[pl.ds(off, tail)], t_idx_d)
            pltpu.async_copy(z_hbm.at[t_idx_s], t_rows, sem).wait()
            pltpu.sync_copy(t_rows, acc.at[t_idx_d], add=True)

        plsc.subcore_barrier()

        @pl.when(s == 0)
        def _():
            pltpu.sync_copy(acc, part_hbm.at[c])

    return shift


def _row_grid(N):
    BM = 1000 if N % 1000 == 0 else 8
    return N // BM, BM


def _readin(x, W, b):
    N, D_in = x.shape
    D = W.shape[1]
    G, BM = _row_grid(N)

    def body(x_ref, w_ref, b_ref, h_ref):
        h = jnp.dot(x_ref[...], w_ref[...],
                    preferred_element_type=jnp.float32) + b_ref[...]
        h_ref[...] = _leaky(h)

    return pl.pallas_call(
        body,
        grid=(G,),
        in_specs=[
            pl.BlockSpec((BM, D_in), lambda i: (i, 0)),
            pl.BlockSpec((D_in, D), lambda i: (0, 0)),
            pl.BlockSpec((1, D), lambda i: (0, 0)),
        ],
        out_specs=pl.BlockSpec((BM, D), lambda i: (i, 0)),
        out_shape=jax.ShapeDtypeStruct((N, D), jnp.float32),
    )(x, W, b)


def _pre(h, W, b):
    N, D = h.shape
    G, BM = _row_grid(N)

    def body(h_ref, w_ref, b_ref, y_ref, o_ref):
        y = _leaky(h_ref[...])
        y_ref[...] = y
        o_ref[...] = jnp.dot(y, w_ref[...],
                             preferred_element_type=jnp.float32) + b_ref[...]

    return pl.pallas_call(
        body,
        grid=(G,),
        in_specs=[
            pl.BlockSpec((BM, D), lambda i: (i, 0)),
            pl.BlockSpec((D, D), lambda i: (0, 0)),
            pl.BlockSpec((1, D), lambda i: (0, 0)),
        ],
        out_specs=[
            pl.BlockSpec((BM, D), lambda i: (i, 0)),
            pl.BlockSpec((BM, D), lambda i: (i, 0)),
        ],
        out_shape=[
            jax.ShapeDtypeStruct((N, D), jnp.float32),
            jax.ShapeDtypeStruct((N, D), jnp.float32),
        ],
    )(h, W, b)


def _tap(p, W, b, out_in):
    _, N, D = p.shape
    G, BM = _row_grid(N)

    def body(p_ref, w_ref, b_ref, oin_ref, z_ref, o_ref):
        z = p_ref[0] + p_ref[1]
        z_ref[...] = z
        o_ref[...] = oin_ref[...] + jnp.dot(
            z, w_ref[...], preferred_element_type=jnp.float32) + b_ref[...]

    return pl.pallas_call(
        body,
        grid=(G,),
        in_specs=[
            pl.BlockSpec((2, BM, D), lambda i: (0, i, 0)),
            pl.BlockSpec((D, D), lambda i: (0, 0)),
            pl.BlockSpec((1, D), lambda i: (0, 0)),
            pl.BlockSpec((BM, D), lambda i: (i, 0)),
        ],
        out_specs=[
            pl.BlockSpec((BM, D), lambda i: (i, 0)),
            pl.BlockSpec((BM, D), lambda i: (i, 0)),
        ],
        out_shape=[
            jax.ShapeDtypeStruct((N, D), jnp.float32),
            jax.ShapeDtypeStruct((N, D), jnp.float32),
        ],
    )(p, W, b, out_in)


def _last(p, W, b, out_in, h):
    _, N, D = p.shape
    G, BM = _row_grid(N)

    def body(p_ref, w_ref, b_ref, oin_ref, h_ref, hn_ref):
        z = p_ref[0] + p_ref[1]
        hn_ref[...] = h_ref[...] + oin_ref[...] + jnp.dot(
            z, w_ref[...], preferred_element_type=jnp.float32) + b_ref[...]

    return pl.pallas_call(
        body,
        grid=(G,),
        in_specs=[
            pl.BlockSpec((2, BM, D), lambda i: (0, i, 0)),
            pl.BlockSpec((D, D), lambda i: (0, 0)),
            pl.BlockSpec((1, D), lambda i: (0, 0)),
            pl.BlockSpec((BM, D), lambda i: (i, 0)),
            pl.BlockSpec((BM, D), lambda i: (i, 0)),
        ],
        out_specs=pl.BlockSpec((BM, D), lambda i: (i, 0)),
        out_shape=jax.ShapeDtypeStruct((N, D), jnp.float32),
    )(p, W, b, out_in, h)


def _readout(h, W, b):
    N, D = h.shape
    D_out = W.shape[1]
    G, BM = _row_grid(N)

    def body(h_ref, w_ref, b_ref, o_ref):
        o_ref[...] = jnp.dot(h_ref[...], w_ref[...],
                             preferred_element_type=jnp.float32) + b_ref[...]

    return pl.pallas_call(
        body,
        grid=(G,),
        in_specs=[
            pl.BlockSpec((BM, D), lambda i: (i, 0)),
            pl.BlockSpec((D, D_out), lambda i: (0, 0)),
            pl.BlockSpec((1, D_out), lambda i: (0, 0)),
        ],
        out_specs=pl.BlockSpec((BM, D_out), lambda i: (i, 0)),
        out_shape=jax.ShapeDtypeStruct((N, D_out), jnp.float32),
    )(h, W, b)


def kernel(x, edge_index, W_in, b_in, taps_W, taps_b, W_out, b_out):
    N = x.shape[0]
    D = W_in.shape[1]
    E = edge_index.shape[1]
    L, T1 = taps_W.shape[0], taps_W.shape[1]

    src = edge_index[0]
    dst = edge_index[1]
    zeros = jnp.zeros((N, D), jnp.float32)
    shift = _make_shift(N, E, D)

    h = _readin(x, W_in, b_in.reshape(1, D))
    for l in range(L):
        z, out = _pre(h, taps_W[l, 0], taps_b[l, 0].reshape(1, D))
        for t in range(1, T1):
            p = shift(z, src, dst, zeros)
            if t < T1 - 1:
                z, out = _tap(p, taps_W[l, t], taps_b[l, t].reshape(1, D), out)
            else:
                h = _last(p, taps_W[l, t], taps_b[l, t].reshape(1, D), out, h)
    return _readout(h, W_out, b_out.reshape(1, W_out.shape[1]))


# speedup vs baseline: 18.6289x; 2.7631x over previous
import functools

import jax
import jax.numpy as jnp
from jax import lax
from jax.experimental import pallas as pl
from jax.experimental.pallas import tpu as pltpu
from jax.experimental.pallas import tpu_sc as plsc

_SLOPE = 0.01


def _leaky(v):
    return jnp.where(v >= 0, v, _SLOPE * v)


_CH = 128
_NB = 8
_PAD_ROWS = 8


def _sc_geometry(E):
    try:
        info = plsc.get_sparse_core_info()
        NC, NS = info.num_cores, info.num_subcores
    except ValueError:
        NC, NS = 2, 16
    NW = NC * NS
    NCH = -(-E // (NW * _CH))
    return NC, NS, NW, NCH


@functools.lru_cache(maxsize=None)
def _make_shift(N, E, D):
    NC, NS, NW, NCH = _sc_geometry(E)
    NP = N + _PAD_ROWS

    mesh = plsc.VectorSubcoreMesh(core_axis_name="c", subcore_axis_name="s",
                                  num_cores=NC, num_subcores=NS)

    scratch = [
        pltpu.VMEM((NCH, 2, _CH), jnp.int32),
        pltpu.VMEM((_NB, _CH, D), jnp.float32),
        pltpu.VMEM_SHARED((NP, D), jnp.float32),
        pltpu.SemaphoreType.DMA((_NB,)),
        pltpu.SemaphoreType.DMA((_NB,)),
    ]

    @functools.partial(
        pl.kernel,
        out_type=jax.ShapeDtypeStruct((NC, N, D), jnp.float32),
        mesh=mesh,
        scratch_types=scratch,
        compiler_params=pltpu.CompilerParams(use_tc_tiling_on_sc=False),
    )
    def shift(z_hbm, edges_hbm, zeros_hbm, part_hbm, idx, rows, acc,
              sem_g, sem_s):
        c = lax.axis_index("c")
        s = lax.axis_index("s")
        wid = c * NS + s

        pltpu.sync_copy(edges_hbm.at[pl.ds(wid * NCH, NCH)], idx)

        @pl.when(s == 0)
        def _():
            pltpu.sync_copy(zeros_hbm, acc)
        plsc.subcore_barrier()

        def start_gather(k, b):
            return pltpu.async_copy(z_hbm.at[idx.at[k, 0]], rows.at[b],
                                    sem_g.at[b])

        def start_scatter(k, b):
            return pltpu.async_copy(rows.at[b], acc.at[idx.at[k, 1]],
                                    sem_s.at[b], add=True)

        def wait_scatter(b):
            pltpu.make_async_copy(rows.at[b], acc.at[idx.at[0, 1]],
                                  sem_s.at[b]).wait()

        NG = NCH // _NB
        TAIL = NCH - NG * _NB

        def body(g, carry):
            descs = []
            for b in range(_NB):
                @pl.when(g > 0)
                def _(b=b):
                    wait_scatter(b)
                descs.append(start_gather(g * _NB + b, b))
            for b in range(_NB):
                descs[b].wait()
                start_scatter(g * _NB + b, b)
            return carry

        lax.fori_loop(0, NG, body, 0)

        tdescs = []
        for b in range(TAIL):
            if NG > 0:
                wait_scatter(b)
            tdescs.append(start_gather(NG * _NB + b, b))
        for b in range(TAIL):
            tdescs[b].wait()
            start_scatter(NG * _NB + b, b)
        for b in range(_NB):
            if b < TAIL or NG > 0:
                wait_scatter(b)

        plsc.subcore_barrier()

        @pl.when(s == 0)
        def _():
            pltpu.sync_copy(acc.at[pl.ds(0, N)], part_hbm.at[c])

    return shift


def _row_grid(N):
    BM = 1000 if N % 1000 == 0 else 8
    return N // BM, BM


def _readin(x, W, b):
    N, D_in = x.shape
    D = W.shape[1]
    G, BM = _row_grid(N)

    def body(x_ref, w_ref, b_ref, h_ref):
        h = jnp.dot(x_ref[...], w_ref[...],
                    preferred_element_type=jnp.float32) + b_ref[...]
        h_ref[...] = _leaky(h)

    return pl.pallas_call(
        body,
        grid=(G,),
        in_specs=[
            pl.BlockSpec((BM, D_in), lambda i: (i, 0)),
            pl.BlockSpec((D_in, D), lambda i: (0, 0)),
            pl.BlockSpec((1, D), lambda i: (0, 0)),
        ],
        out_specs=pl.BlockSpec((BM, D), lambda i: (i, 0)),
        out_shape=jax.ShapeDtypeStruct((N, D), jnp.float32),
    )(x, W, b)


def _pre(h, W, b):
    N, D = h.shape
    G, BM = _row_grid(N)

    def body(h_ref, w_ref, b_ref, y_ref, o_ref):
        y = _leaky(h_ref[...])
        y_ref[...] = y
        o_ref[...] = jnp.dot(y, w_ref[...],
                             preferred_element_type=jnp.float32) + b_ref[...]

    return pl.pallas_call(
        body,
        grid=(G,),
        in_specs=[
            pl.BlockSpec((BM, D), lambda i: (i, 0)),
            pl.BlockSpec((D, D), lambda i: (0, 0)),
            pl.BlockSpec((1, D), lambda i: (0, 0)),
        ],
        out_specs=[
            pl.BlockSpec((BM, D), lambda i: (i, 0)),
            pl.BlockSpec((BM, D), lambda i: (i, 0)),
        ],
        out_shape=[
            jax.ShapeDtypeStruct((N, D), jnp.float32),
            jax.ShapeDtypeStruct((N, D), jnp.float32),
        ],
    )(h, W, b)


def _tap(p, W, b, out_in):
    _, N, D = p.shape
    G, BM = _row_grid(N)

    def body(p_ref, w_ref, b_ref, oin_ref, z_ref, o_ref):
        z = p_ref[0] + p_ref[1]
        z_ref[...] = z
        o_ref[...] = oin_ref[...] + jnp.dot(
            z, w_ref[...], preferred_element_type=jnp.float32) + b_ref[...]

    return pl.pallas_call(
        body,
        grid=(G,),
        in_specs=[
            pl.BlockSpec((2, BM, D), lambda i: (0, i, 0)),
            pl.BlockSpec((D, D), lambda i: (0, 0)),
            pl.BlockSpec((1, D), lambda i: (0, 0)),
            pl.BlockSpec((BM, D), lambda i: (i, 0)),
        ],
        out_specs=[
            pl.BlockSpec((BM, D), lambda i: (i, 0)),
            pl.BlockSpec((BM, D), lambda i: (i, 0)),
        ],
        out_shape=[
            jax.ShapeDtypeStruct((N, D), jnp.float32),
            jax.ShapeDtypeStruct((N, D), jnp.float32),
        ],
    )(p, W, b, out_in)


def _last(p, W, b, out_in, h):
    _, N, D = p.shape
    G, BM = _row_grid(N)

    def body(p_ref, w_ref, b_ref, oin_ref, h_ref, hn_ref):
        z = p_ref[0] + p_ref[1]
        hn_ref[...] = h_ref[...] + oin_ref[...] + jnp.dot(
            z, w_ref[...], preferred_element_type=jnp.float32) + b_ref[...]

    return pl.pallas_call(
        body,
        grid=(G,),
        in_specs=[
            pl.BlockSpec((2, BM, D), lambda i: (0, i, 0)),
            pl.BlockSpec((D, D), lambda i: (0, 0)),
            pl.BlockSpec((1, D), lambda i: (0, 0)),
            pl.BlockSpec((BM, D), lambda i: (i, 0)),
            pl.BlockSpec((BM, D), lambda i: (i, 0)),
        ],
        out_specs=pl.BlockSpec((BM, D), lambda i: (i, 0)),
        out_shape=jax.ShapeDtypeStruct((N, D), jnp.float32),
    )(p, W, b, out_in, h)


def _readout(h, W, b):
    N, D = h.shape
    D_out = W.shape[1]
    G, BM = _row_grid(N)

    def body(h_ref, w_ref, b_ref, o_ref):
        o_ref[...] = jnp.dot(h_ref[...], w_ref[...],
                             preferred_element_type=jnp.float32) + b_ref[...]

    return pl.pallas_call(
        body,
        grid=(G,),
        in_specs=[
            pl.BlockSpec((BM, D), lambda i: (i, 0)),
            pl.BlockSpec((D, D_out), lambda i: (0, 0)),
            pl.BlockSpec((1, D_out), lambda i: (0, 0)),
        ],
        out_specs=pl.BlockSpec((BM, D_out), lambda i: (i, 0)),
        out_shape=jax.ShapeDtypeStruct((N, D_out), jnp.float32),
    )(h, W, b)


def kernel(x, edge_index, W_in, b_in, taps_W, taps_b, W_out, b_out):
    N = x.shape[0]
    D = W_in.shape[1]
    E = edge_index.shape[1]
    L, T1 = taps_W.shape[0], taps_W.shape[1]

    NC, NS, NW, NCH = _sc_geometry(E)
    E_pad = NW * NCH * _CH
    pad = E_pad - E
    if pad:
        ar = jnp.arange(pad, dtype=jnp.int32)
        src = jnp.concatenate([edge_index[0], ar % N])
        dst = jnp.concatenate([edge_index[1], N + (ar % _PAD_ROWS)])
    else:
        src, dst = edge_index[0], edge_index[1]
    edges3 = jnp.stack([src, dst]).reshape(2, NW * NCH, _CH).transpose(1, 0, 2)
    zeros = jnp.zeros((N + _PAD_ROWS, D), jnp.float32)
    shift = _make_shift(N, E, D)

    h = _readin(x, W_in, b_in.reshape(1, D))
    for l in range(L):
        z, out = _pre(h, taps_W[l, 0], taps_b[l, 0].reshape(1, D))
        for t in range(1, T1):
            p = shift(z, edges3, zeros)
            if t < T1 - 1:
                z, out = _tap(p, taps_W[l, t], taps_b[l, t].reshape(1, D), out)
            else:
                h = _last(p, taps_W[l, t], taps_b[l, t].reshape(1, D), out, h)
    return _readout(h, W_out, b_out.reshape(1, W_out.shape[1]))
